# Initial kernel scaffold; baseline (speedup 1.0000x reference)
import jax
import jax.numpy as jnp
from jax.experimental import pallas as pl


def kernel(input, target):
    raise NotImplementedError("write your pallas kernel here")


# speedup vs baseline: 1.6769x; 1.6769x over previous
import jax
import jax.numpy as jnp
from jax.experimental import pallas as pl
from jax.experimental.pallas import tpu as pltpu


def _chamfer_body(a_ref, b_ref, out_ref):
    a = a_ref[0]
    b = b_ref[0]
    ab = jax.lax.dot_general(a, b, (((1,), (1,)), ((), ())),
                             preferred_element_type=jnp.float32)
    na = jnp.sum(a * a, axis=1)
    nb = jnp.sum(b * b, axis=1)
    d2 = (na[:, None] - 2.0 * ab) + nb[None, :]
    m_b = jnp.min(d2, axis=0)
    m_a = jnp.min(d2, axis=1)
    loss = (jnp.mean(jnp.sqrt(jnp.maximum(m_b, 1e-12)))
            + jnp.mean(jnp.sqrt(jnp.maximum(m_a, 1e-12))))
    out_ref[...] = jnp.full((1, 1, 128), loss, jnp.float32)


def kernel(input, target):
    B, N, _ = input.shape
    losses = pl.pallas_call(
        _chamfer_body,
        grid=(B,),
        in_specs=[
            pl.BlockSpec((1, N, 3), lambda i: (i, 0, 0)),
            pl.BlockSpec((1, N, 3), lambda i: (i, 0, 0)),
        ],
        out_specs=pl.BlockSpec((1, 1, 128), lambda i: (i, 0, 0)),
        out_shape=jax.ShapeDtypeStruct((B, 1, 128), jnp.float32),
    )(input, target)
    return jnp.reshape(jnp.sum(losses[:, 0, 0]), (1,))
